# baseline (speedup 1.0000x reference)
import math
import functools

import jax
import jax.numpy as jnp
from jax import lax
from jax.experimental import pallas as pl
from jax.experimental.pallas import tpu as pltpu
from jax.experimental.pallas import tpu_sc as plsc

N = 10000
E = 320000
H = 128
G = 8
HG = H // G
ROW = 3 * HG
OC = 4 * HG
OW = 128
NC = 2
NS = 16
EPT = E // NS
K = 80
NBLK = EPT // K
NPT = 624
NTAIL = N - NS * NPT
CW = 128


def _xh_body(x_ref, vg_ref, g_ref, b_ref, w1_ref, b1_ref, w2_ref, b2_ref, o_ref):
    x = x_ref[...]
    mu = jnp.mean(x, axis=1, keepdims=True)
    var = jnp.mean((x - mu) ** 2, axis=1, keepdims=True)
    xln = (x - mu) * lax.rsqrt(var + 1e-5) * g_ref[...] + b_ref[...]
    t = jnp.dot(xln, w1_ref[...], preferred_element_type=jnp.float32) + b1_ref[...]
    s = t * jax.nn.sigmoid(t) * (1.0 / 0.6)
    xh = jnp.dot(s, w2_ref[0], preferred_element_type=jnp.float32) + b2_ref[0]
    pad = jnp.zeros((x.shape[0], CW - 2 * ROW), jnp.float32)
    o_ref[0] = jnp.concatenate([xh, vg_ref[0], pad], axis=1)


def _node_table_grouped(x, vec_g, ln_gamma, ln_beta, W1, b1, W2p, b2p):
    bn = 1000
    return pl.pallas_call(
        _xh_body,
        grid=(G, N // bn),
        in_specs=[
            pl.BlockSpec((bn, H), lambda g, i: (i, 0)),
            pl.BlockSpec((1, bn, ROW), lambda g, i: (g, i, 0)),
            pl.BlockSpec((1, H), lambda g, i: (0, 0)),
            pl.BlockSpec((1, H), lambda g, i: (0, 0)),
            pl.BlockSpec((H, H), lambda g, i: (0, 0)),
            pl.BlockSpec((1, H), lambda g, i: (0, 0)),
            pl.BlockSpec((1, H, ROW), lambda g, i: (g, 0, 0)),
            pl.BlockSpec((1, 1, ROW), lambda g, i: (g, 0, 0)),
        ],
        out_specs=pl.BlockSpec((1, bn, CW), lambda g, i: (g, i, 0)),
        out_shape=jax.ShapeDtypeStruct((G, N, CW), jnp.float32),
    )(x, vec_g, ln_gamma.reshape(1, H), ln_beta.reshape(1, H),
      W1, b1.reshape(1, H), W2p, b2p)


def _rbf_body(r_ref, wr_ref, br_ref, o_ref):
    y = (jnp.dot(r_ref[...], wr_ref[...], preferred_element_type=jnp.float32)
         + br_ref[...])
    for g in range(G):
        o_ref[g] = y[:, g * ROW:(g + 1) * ROW]


def _rbf_mlp_grouped(edge_rbf, Wrp, brp):
    r = edge_rbf.shape[1]
    be = 2000
    return pl.pallas_call(
        _rbf_body,
        grid=(E // be,),
        in_specs=[
            pl.BlockSpec((be, r), lambda i: (i, 0)),
            pl.BlockSpec((r, G * ROW), lambda i: (0, 0)),
            pl.BlockSpec((1, G * ROW), lambda i: (0, 0)),
        ],
        out_specs=pl.BlockSpec((G, be, ROW), lambda i: (0, i, 0)),
        out_shape=jax.ShapeDtypeStruct((G, E, ROW), jnp.float32),
    )(edge_rbf, Wrp, brp.reshape(1, G * ROW))


STEADY = (NBLK - 4) // 3


M = 2


def _sc_body(comb, rbfg, src_h, dst_h, ev_h, zeros_h, out_h,
             srcb0, srcb1, dstb0, cb0, cb1, rb0, eb0,
             outb, acc,
             sems0, sems1, semg0, semg1):
    c = lax.axis_index("c")
    t = lax.axis_index("s")
    srcb = [srcb0, srcb1]
    dstb = [dstb0, dstb0]
    cbuf = [cb0, cb1]
    rbb = [rb0, rb0]
    evb = [eb0, eb0]
    sems = [sems0, sems1]
    semg = [semg0, semg1]

    @pl.loop(0, K)
    def _zpad(j):
        for kk in range((OW - OC) // 16):
            outb[j, pl.ds(OC + kk * 16, 16)] = jnp.zeros((16,), jnp.float32)

    for gi in range(G // NC):
        g = (G // NC) * c + gi

        pltpu.sync_copy(zeros_h.at[pl.ds(t * NPT, NPT)],
                        acc.at[pl.ds(t * NPT, NPT)])

        @pl.when(t == 0)
        def _zero_tail():
            pltpu.sync_copy(zeros_h.at[pl.ds(NS * NPT, NTAIL)],
                            acc.at[pl.ds(NS * NPT, NTAIL)])

        plsc.subcore_barrier()

        def compute_scatter(s):
            cb = cbuf[s]
            rb = rbb[s]
            eb = evb[s]

            @pl.loop(0, K, unroll=2)
            def _edge(j):
                ev = eb[pl.ds(3 * j, 16)]
                ev0 = ev[0]
                ev1 = ev[1]
                ev2 = ev[2]
                for k in range(HG // 16):
                    o = k * 16
                    outb[j, pl.ds(o, 16)] = cb[j, pl.ds(o, 16)] * rb[j, pl.ds(o, 16)]
                    m2 = cb[j, pl.ds(HG + o, 16)] * rb[j, pl.ds(HG + o, 16)]
                    m3 = cb[j, pl.ds(2 * HG + o, 16)] * rb[j, pl.ds(2 * HG + o, 16)]
                    outb[j, pl.ds(HG + o, 16)] = cb[j, pl.ds(ROW + o, 16)] * m2 + m3 * ev0
                    outb[j, pl.ds(2 * HG + o, 16)] = cb[j, pl.ds(ROW + HG + o, 16)] * m2 + m3 * ev1
                    outb[j, pl.ds(3 * HG + o, 16)] = cb[j, pl.ds(ROW + 2 * HG + o, 16)] * m2 + m3 * ev2

            pltpu.sync_copy(outb, acc.at[dstb[s]], add=True)

        @pl.loop(0, NBLK // M)
        def _super(i):
            b0 = i * M
            ds_src = []
            for m in range(M):
                e0 = t * EPT + (b0 + m) * K
                ds_src.append(
                    pltpu.async_copy(src_h.at[pl.ds(e0, K)], srcb[m], sems[m]))
            for m in range(M):
                ds_src[m].wait()
            ds_g = []
            for m in range(M):
                @pl.loop(0, K // 16)
                def _adj(i2):
                    srcb[m][pl.ds(i2 * 16, 16)] = (
                        srcb[m][pl.ds(i2 * 16, 16)] + g * N)

                ds_g.append(
                    pltpu.async_copy(comb.at[srcb[m]], cbuf[m], semg[m]))
            for m in range(M):
                e0 = t * EPT + (b0 + m) * K
                pltpu.sync_copy(dst_h.at[pl.ds(e0, K)], dstb[m])
                pltpu.sync_copy(rbfg.at[pl.ds(g * E + e0, K)], rbb[m])
                pltpu.sync_copy(ev_h.at[pl.ds(3 * e0, 3 * K)],
                                evb[m].at[pl.ds(0, 3 * K)])
                ds_g[m].wait()
                compute_scatter(m)

        plsc.subcore_barrier()

        pltpu.sync_copy(acc.at[pl.ds(t * NPT, NPT)],
                        out_h.at[pl.ds(g * N + t * NPT, NPT)])

        @pl.when(t == 0)
        def _flush_tail():
            pltpu.sync_copy(acc.at[pl.ds(NS * NPT, NTAIL)],
                            out_h.at[pl.ds(g * N + NS * NPT, NTAIL)])


@functools.cache
def _sc_aggregate():
    return pl.kernel(
        _sc_body,
        out_type=jax.ShapeDtypeStruct((G * N, OW), jnp.float32),
        mesh=plsc.VectorSubcoreMesh(core_axis_name="c", subcore_axis_name="s",
                                    num_cores=NC, num_subcores=NS),
        scratch_types=(
        [pltpu.VMEM((K,), jnp.int32)] * 2
        + [pltpu.VMEM((K,), jnp.int32)] * 1
        + [pltpu.VMEM((K, CW), jnp.float32)] * 2
        + [pltpu.VMEM((K, ROW), jnp.float32)] * 1
        + [pltpu.VMEM((3 * K + 16,), jnp.float32)] * 1
        + [
            pltpu.VMEM((K, OW), jnp.float32),
            pltpu.VMEM_SHARED((N, OW), jnp.float32),
        ]
        + [pltpu.SemaphoreType.DMA] * 4
        ),
    )


def kernel(x, vec, edge_index, edge_rbf, edge_vector,
           ln_gamma, ln_beta, W1, b1, W2, b2, Wr, br):
    perm = []
    for g in range(G):
        for part in range(3):
            perm.extend(range(part * H + g * HG, part * H + (g + 1) * HG))
    perm = jnp.asarray(perm, dtype=jnp.int32)
    scale = jnp.concatenate([
        jnp.full((H,), 1.0, jnp.float32),
        jnp.full((H,), 1.0 / (math.sqrt(3.0) * math.sqrt(H)), jnp.float32),
        jnp.full((H,), 1.0 / math.sqrt(H), jnp.float32),
    ])
    W2p = W2[:, perm].reshape(H, G, ROW).transpose(1, 0, 2)
    b2p = b2[perm].reshape(G, 1, ROW)
    Wrp = (Wr * scale[None, :])[:, perm]
    brp = (br * scale)[perm]

    vec_g = vec.reshape(N, 3, G, HG).transpose(2, 0, 1, 3).reshape(G, N, ROW)
    comb = _node_table_grouped(x, vec_g, ln_gamma, ln_beta, W1, b1, W2p, b2p)
    rbf_g = _rbf_mlp_grouped(edge_rbf, Wrp, brp)

    src = edge_index[0]
    dst = edge_index[1]
    ev_flat = edge_vector.reshape(3 * E)

    acc = _sc_aggregate()(
        comb.reshape(G * N, CW), rbf_g.reshape(G * E, ROW),
        src, dst, ev_flat, jnp.zeros((N, OW), jnp.float32),
    )

    acc = acc.reshape(G, N, OW)
    dx = acc[:, :, :HG].transpose(1, 0, 2).reshape(N, H)
    dvec = (acc[:, :, HG:OC].reshape(G, N, 3, HG)
            .transpose(1, 2, 0, 3).reshape(N, 3, H))
    return (dx, dvec)


# speedup vs baseline: 1.4853x; 1.4853x over previous
import math
import functools

import jax
import jax.numpy as jnp
from jax import lax
from jax.experimental import pallas as pl
from jax.experimental.pallas import tpu as pltpu
from jax.experimental.pallas import tpu_sc as plsc

N = 10000
E = 320000
H = 128
G = 4
HG = H // G
ROW = 3 * HG
OC = 4 * HG
OW = OC
NC = 2
NS = 16
EPT = E // NS
K = 80
NBLK = EPT // K
KH = K // 2
NPT = 624
NTAIL = N - NS * NPT
CW = 256


def _xh_body(x_ref, vg_ref, g_ref, b_ref, w1_ref, b1_ref, w2_ref, b2_ref, o_ref):
    x = x_ref[...]
    mu = jnp.mean(x, axis=1, keepdims=True)
    var = jnp.mean((x - mu) ** 2, axis=1, keepdims=True)
    xln = (x - mu) * lax.rsqrt(var + 1e-5) * g_ref[...] + b_ref[...]
    t = jnp.dot(xln, w1_ref[...], preferred_element_type=jnp.float32) + b1_ref[...]
    s = t * jax.nn.sigmoid(t) * (1.0 / 0.6)
    xh = jnp.dot(s, w2_ref[0], preferred_element_type=jnp.float32) + b2_ref[0]
    pad = jnp.zeros((x.shape[0], CW - 2 * ROW), jnp.float32)
    o_ref[0] = jnp.concatenate([xh, vg_ref[0], pad], axis=1)


def _node_table_grouped(x, vec_g, ln_gamma, ln_beta, W1, b1, W2p, b2p):
    bn = 1000
    return pl.pallas_call(
        _xh_body,
        grid=(G, N // bn),
        in_specs=[
            pl.BlockSpec((bn, H), lambda g, i: (i, 0)),
            pl.BlockSpec((1, bn, ROW), lambda g, i: (g, i, 0)),
            pl.BlockSpec((1, H), lambda g, i: (0, 0)),
            pl.BlockSpec((1, H), lambda g, i: (0, 0)),
            pl.BlockSpec((H, H), lambda g, i: (0, 0)),
            pl.BlockSpec((1, H), lambda g, i: (0, 0)),
            pl.BlockSpec((1, H, ROW), lambda g, i: (g, 0, 0)),
            pl.BlockSpec((1, 1, ROW), lambda g, i: (g, 0, 0)),
        ],
        out_specs=pl.BlockSpec((1, bn, CW), lambda g, i: (g, i, 0)),
        out_shape=jax.ShapeDtypeStruct((G, N, CW), jnp.float32),
    )(x, vec_g, ln_gamma.reshape(1, H), ln_beta.reshape(1, H),
      W1, b1.reshape(1, H), W2p, b2p)


def _rbf_body(r_ref, wr_ref, br_ref, o_ref):
    y = (jnp.dot(r_ref[...], wr_ref[...], preferred_element_type=jnp.float32)
         + br_ref[...])
    for g in range(G):
        o_ref[g] = y[:, g * ROW:(g + 1) * ROW]


def _rbf_mlp_grouped(edge_rbf, Wrp, brp):
    r = edge_rbf.shape[1]
    be = 2000
    return pl.pallas_call(
        _rbf_body,
        grid=(E // be,),
        in_specs=[
            pl.BlockSpec((be, r), lambda i: (i, 0)),
            pl.BlockSpec((r, G * ROW), lambda i: (0, 0)),
            pl.BlockSpec((1, G * ROW), lambda i: (0, 0)),
        ],
        out_specs=pl.BlockSpec((G, be, ROW), lambda i: (0, i, 0)),
        out_shape=jax.ShapeDtypeStruct((G, E, ROW), jnp.float32),
    )(edge_rbf, Wrp, brp.reshape(1, G * ROW))


def _sc_body(comb, rbfg, src_h, dst_h, ev_h, zeros_h, out_h,
             srcb, dstb, dst1, dst2, cb1, cb2, rbb, evb,
             outb, acc,
             semsrc, semdst, semrb, semev, semg1, semg2):
    c = lax.axis_index("c")
    t = lax.axis_index("s")

    for j in range(KH, KH + 8):
        for kk in range(OW // 16):
            outb[j, pl.ds(kk * 16, 16)] = jnp.zeros((16,), jnp.float32)
    dstb[pl.ds(K, 16)] = jnp.zeros((16,), jnp.int32)

    for gi in range(G // NC):
        g = (G // NC) * c + gi

        pltpu.sync_copy(zeros_h.at[pl.ds(t * NPT, NPT)],
                        acc.at[pl.ds(t * NPT, NPT)])

        @pl.when(t == 0)
        def _zero_tail():
            pltpu.sync_copy(zeros_h.at[pl.ds(NS * NPT, NTAIL)],
                            acc.at[pl.ds(NS * NPT, NTAIL)])

        plsc.subcore_barrier()

        def compute_half(cb, base, dsth):
            @pl.loop(0, KH, unroll=2)
            def _edge(j):
                ev = evb[pl.ds(3 * (base + j), 16)]
                ev0 = ev[0]
                ev1 = ev[1]
                ev2 = ev[2]
                for k in range(HG // 16):
                    o = k * 16
                    outb[j, pl.ds(o, 16)] = (cb[j, pl.ds(o, 16)]
                                             * rbb[base + j, pl.ds(o, 16)])
                    m2 = cb[j, pl.ds(HG + o, 16)] * rbb[base + j, pl.ds(HG + o, 16)]
                    m3 = cb[j, pl.ds(2 * HG + o, 16)] * rbb[base + j, pl.ds(2 * HG + o, 16)]
                    outb[j, pl.ds(HG + o, 16)] = cb[j, pl.ds(ROW + o, 16)] * m2 + m3 * ev0
                    outb[j, pl.ds(2 * HG + o, 16)] = cb[j, pl.ds(ROW + HG + o, 16)] * m2 + m3 * ev1
                    outb[j, pl.ds(3 * HG + o, 16)] = cb[j, pl.ds(ROW + 2 * HG + o, 16)] * m2 + m3 * ev2

            pltpu.sync_copy(outb, acc.at[dsth], add=True)

        @pl.loop(0, NBLK)
        def _block(b):
            e0 = t * EPT + b * K
            d_src = pltpu.async_copy(src_h.at[pl.ds(e0, K)], srcb.at[pl.ds(0, K)], semsrc)
            d_dst = pltpu.async_copy(dst_h.at[pl.ds(e0, K)], dstb.at[pl.ds(0, K)], semdst)
            d_rb = pltpu.async_copy(rbfg.at[pl.ds(g * E + e0, K)], rbb, semrb)
            d_ev = pltpu.async_copy(ev_h.at[pl.ds(3 * e0, 3 * K)],
                                    evb.at[pl.ds(0, 3 * K)], semev)
            d_src.wait()

            @pl.loop(0, K // 16)
            def _adj(i2):
                srcb[pl.ds(i2 * 16, 16)] = srcb[pl.ds(i2 * 16, 16)] + g * N

            g1 = pltpu.async_copy(comb.at[srcb.at[pl.ds(0, KH)]], cb1, semg1)
            g2 = pltpu.async_copy(comb.at[srcb.at[pl.ds(KH, KH)]], cb2, semg2)

            d_dst.wait()
            for i2 in range(3):
                dst1[pl.ds(i2 * 16, 16)] = dstb[pl.ds(i2 * 16, 16)]
                dst2[pl.ds(i2 * 16, 16)] = dstb[pl.ds(KH + i2 * 16, 16)]
            d_rb.wait()
            d_ev.wait()

            g1.wait()
            compute_half(cb1, 0, dst1)
            g2.wait()
            compute_half(cb2, KH, dst2)

        plsc.subcore_barrier()

        pltpu.sync_copy(acc.at[pl.ds(t * NPT, NPT)],
                        out_h.at[pl.ds(g * N + t * NPT, NPT)])

        @pl.when(t == 0)
        def _flush_tail():
            pltpu.sync_copy(acc.at[pl.ds(NS * NPT, NTAIL)],
                            out_h.at[pl.ds(g * N + NS * NPT, NTAIL)])


@functools.cache
def _sc_aggregate():
    return pl.kernel(
        _sc_body,
        out_type=jax.ShapeDtypeStruct((G * N, OW), jnp.float32),
        mesh=plsc.VectorSubcoreMesh(core_axis_name="c", subcore_axis_name="s",
                                    num_cores=NC, num_subcores=NS),
        scratch_types=(
        [
            pltpu.VMEM((K,), jnp.int32),
            pltpu.VMEM((K + 16,), jnp.int32),
            pltpu.VMEM((KH + 8,), jnp.int32),
            pltpu.VMEM((KH + 8,), jnp.int32),
            pltpu.VMEM((KH, CW), jnp.float32),
            pltpu.VMEM((KH, CW), jnp.float32),
            pltpu.VMEM((K, ROW), jnp.float32),
            pltpu.VMEM((3 * K + 16,), jnp.float32),
            pltpu.VMEM((KH + 8, OW), jnp.float32),
            pltpu.VMEM_SHARED((N, OW), jnp.float32),
        ]
        + [pltpu.SemaphoreType.DMA] * 6
        ),
    )


def kernel(x, vec, edge_index, edge_rbf, edge_vector,
           ln_gamma, ln_beta, W1, b1, W2, b2, Wr, br):
    perm = []
    for g in range(G):
        for part in range(3):
            perm.extend(range(part * H + g * HG, part * H + (g + 1) * HG))
    perm = jnp.asarray(perm, dtype=jnp.int32)
    scale = jnp.concatenate([
        jnp.full((H,), 1.0, jnp.float32),
        jnp.full((H,), 1.0 / (math.sqrt(3.0) * math.sqrt(H)), jnp.float32),
        jnp.full((H,), 1.0 / math.sqrt(H), jnp.float32),
    ])
    W2p = W2[:, perm].reshape(H, G, ROW).transpose(1, 0, 2)
    b2p = b2[perm].reshape(G, 1, ROW)
    Wrp = (Wr * scale[None, :])[:, perm]
    brp = (br * scale)[perm]

    vec_g = vec.reshape(N, 3, G, HG).transpose(2, 0, 1, 3).reshape(G, N, ROW)
    comb = _node_table_grouped(x, vec_g, ln_gamma, ln_beta, W1, b1, W2p, b2p)
    rbf_g = _rbf_mlp_grouped(edge_rbf, Wrp, brp)

    src = edge_index[0]
    dst = edge_index[1]
    ev_flat = edge_vector.reshape(3 * E)

    acc = _sc_aggregate()(
        comb.reshape(G * N, CW), rbf_g.reshape(G * E, ROW),
        src, dst, ev_flat, jnp.zeros((N, OW), jnp.float32),
    )

    acc = acc.reshape(G, N, OW)
    dx = acc[:, :, :HG].transpose(1, 0, 2).reshape(N, H)
    dvec = (acc[:, :, HG:OC].reshape(G, N, 3, HG)
            .transpose(1, 2, 0, 3).reshape(N, 3, H))
    return (dx, dvec)
